# baseline (speedup 1.0000x reference)
import functools
import math

import jax
import jax.numpy as jnp
from jax import lax
from jax.experimental import pallas as pl
from jax.experimental.pallas import tpu as pltpu
from jax.experimental.pallas import tpu_sc as plsc

VOCAB = 1000000
EMB = 64
B = 4096
L = 200
SCALE = math.sqrt(EMB)

NC, NS = 2, 16
NW = NC * NS
BFLAT = B * L
PER_W = BFLAT // NW
IROW = 128
K = 8
CHUNK = K * IROW
N_CHUNKS = PER_W // CHUNK
ROWS_PER_W = PER_W // IROW


def _emb_kernel(tok_hbm, table_hbm, out_hbm, idx_v, rows_v, sem):
    wid = lax.axis_index("s") * NC + lax.axis_index("c")
    row0 = wid * ROWS_PER_W
    flat0 = wid * PER_W

    def chunk_body(ci, _):
        pltpu.sync_copy(tok_hbm.at[pl.ds(row0 + ci * K, K)], idx_v)
        descs = [
            pltpu.async_copy(
                table_hbm.at[idx_v.at[j]],
                rows_v.at[pl.ds(j * IROW, IROW)],
                sem,
            )
            for j in range(K)
        ]
        for d in descs:
            d.wait()

        def scale_body(r, _):
            for c in range(EMB // 16):
                sl = pl.ds(c * 16, 16)
                rows_v[r, sl] = rows_v[r, sl] * SCALE
            return 0

        lax.fori_loop(0, CHUNK, scale_body, 0, unroll=4)
        pltpu.sync_copy(rows_v, out_hbm.at[pl.ds(flat0 + ci * CHUNK, CHUNK)])
        return 0

    lax.fori_loop(0, N_CHUNKS, chunk_body, 0)


@jax.jit
def kernel(tokens, table):
    tok2d = tokens.reshape(BFLAT // IROW, IROW).astype(jnp.int32)
    mesh = plsc.VectorSubcoreMesh(core_axis_name="c", subcore_axis_name="s")
    out = pl.kernel(
        _emb_kernel,
        out_type=jax.ShapeDtypeStruct((BFLAT, EMB), jnp.float32),
        mesh=mesh,
        scratch_types=[
            pltpu.VMEM((K, IROW), jnp.int32),
            pltpu.VMEM((CHUNK, EMB), jnp.float32),
            pltpu.SemaphoreType.DMA,
        ],
        compiler_params=pltpu.CompilerParams(use_tc_tiling_on_sc=False),
    )(tok2d, table)
    return out.reshape(B, L, EMB)


# speedup vs baseline: 1.0587x; 1.0587x over previous
import math

import jax
import jax.numpy as jnp
from jax import lax
from jax.experimental import pallas as pl
from jax.experimental.pallas import tpu as pltpu
from jax.experimental.pallas import tpu_sc as plsc

VOCAB = 1000000
EMB = 64
B = 4096
L = 200
SCALE = math.sqrt(EMB)

NC, NS = 2, 16
NW = NC * NS
BFLAT = B * L
PER_W = BFLAT // NW
IROW = 128
K = 5
CHUNK = K * IROW
N_CHUNKS = PER_W // CHUNK
ROWS_PER_W = PER_W // IROW
CHUNK_PAIRS = N_CHUNKS // 2


def _emb_kernel(tok_hbm, table_hbm, out_hbm, idx_v, rows0, rows1, gsem0,
                gsem1, osem0, osem1):
    wid = lax.axis_index("s") * NC + lax.axis_index("c")
    row0 = wid * ROWS_PER_W
    flat0 = wid * PER_W
    rows_b = (rows0, rows1)
    gsem_b = (gsem0, gsem1)
    osem_b = (osem0, osem1)

    pltpu.sync_copy(tok_hbm.at[pl.ds(row0, ROWS_PER_W)], idx_v)

    def fire_gather(c, b):
        for j in range(K):
            pltpu.async_copy(
                table_hbm.at[idx_v.at[c * K + j]],
                rows_b[b].at[pl.ds(j * IROW, IROW)],
                gsem_b[b],
            )

    def drain_gather(b):
        pltpu.make_async_copy(
            out_hbm.at[pl.ds(0, CHUNK)], rows_b[b], gsem_b[b]).wait()

    def drain_scatter(b):
        pltpu.make_async_copy(
            rows_b[b], out_hbm.at[pl.ds(0, CHUNK)], osem_b[b]).wait()

    fire_gather(0, 0)
    fire_gather(1, 1)

    def pair_body(i, _):
        for b in range(2):
            c = 2 * i + b
            drain_gather(b)

            def scale_body(r, _):
                for col in range(EMB // 16):
                    sl = pl.ds(col * 16, 16)
                    rows_b[b][r, sl] = rows_b[b][r, sl] * SCALE
                return 0

            lax.fori_loop(0, CHUNK, scale_body, 0, unroll=4)
            pltpu.async_copy(
                rows_b[b], out_hbm.at[pl.ds(flat0 + c * CHUNK, CHUNK)],
                osem_b[b])

            @pl.when(c + 2 < N_CHUNKS)
            def _():
                drain_scatter(b)
                fire_gather(c + 2, b)

        return 0

    lax.fori_loop(0, CHUNK_PAIRS, pair_body, 0)
    drain_scatter(0)
    drain_scatter(1)


@jax.jit
def kernel(tokens, table):
    tok2d = tokens.reshape(BFLAT // IROW, IROW).astype(jnp.int32)
    mesh = plsc.VectorSubcoreMesh(core_axis_name="c", subcore_axis_name="s")
    out = pl.kernel(
        _emb_kernel,
        out_type=jax.ShapeDtypeStruct((BFLAT, EMB), jnp.float32),
        mesh=mesh,
        scratch_types=[
            pltpu.VMEM((ROWS_PER_W, IROW), jnp.int32),
            pltpu.VMEM((CHUNK, EMB), jnp.float32),
            pltpu.VMEM((CHUNK, EMB), jnp.float32),
            pltpu.SemaphoreType.DMA,
            pltpu.SemaphoreType.DMA,
            pltpu.SemaphoreType.DMA,
            pltpu.SemaphoreType.DMA,
        ],
        compiler_params=pltpu.CompilerParams(use_tc_tiling_on_sc=False),
    )(tok2d, table)
    return out.reshape(B, L, EMB)


# speedup vs baseline: 1.0595x; 1.0007x over previous
import math

import jax
import jax.numpy as jnp
from jax import lax
from jax.experimental import pallas as pl
from jax.experimental.pallas import tpu as pltpu
from jax.experimental.pallas import tpu_sc as plsc

VOCAB = 1000000
EMB = 64
B = 4096
L = 200
SCALE = math.sqrt(EMB)

NC, NS = 2, 16
NW = NC * NS
BFLAT = B * L
PER_W = BFLAT // NW
CHUNK = 640
N_CHUNKS = PER_W // CHUNK
CHUNK_PAIRS = N_CHUNKS // 2


def _emb_kernel(tok_hbm, table_hbm, out_hbm, idx_v, rows0, rows1, gsem0,
                gsem1, osem0, osem1):
    wid = lax.axis_index("s") * NC + lax.axis_index("c")
    flat0 = wid * PER_W
    rows_b = (rows0, rows1)
    gsem_b = (gsem0, gsem1)
    osem_b = (osem0, osem1)

    pltpu.sync_copy(tok_hbm.at[pl.ds(flat0, PER_W)], idx_v)

    def fire_gather(c, b):
        pltpu.async_copy(
            table_hbm.at[idx_v.at[pl.ds(c * CHUNK, CHUNK)]],
            rows_b[b],
            gsem_b[b],
        )

    def drain_gather(b):
        pltpu.make_async_copy(
            out_hbm.at[pl.ds(0, CHUNK)], rows_b[b], gsem_b[b]).wait()

    def drain_scatter(b):
        pltpu.make_async_copy(
            rows_b[b], out_hbm.at[pl.ds(0, CHUNK)], osem_b[b]).wait()

    fire_gather(0, 0)
    fire_gather(1, 1)

    def pair_body(i, _):
        for b in range(2):
            c = 2 * i + b
            drain_gather(b)

            def scale_body(r, _):
                for col in range(EMB // 16):
                    sl = pl.ds(col * 16, 16)
                    rows_b[b][r, sl] = rows_b[b][r, sl] * SCALE
                return 0

            lax.fori_loop(0, CHUNK, scale_body, 0, unroll=4)
            pltpu.async_copy(
                rows_b[b], out_hbm.at[pl.ds(flat0 + c * CHUNK, CHUNK)],
                osem_b[b])

            @pl.when(c + 2 < N_CHUNKS)
            def _():
                drain_scatter(b)
                fire_gather(c + 2, b)

        return 0

    lax.fori_loop(0, CHUNK_PAIRS, pair_body, 0)
    drain_scatter(0)
    drain_scatter(1)


@jax.jit
def kernel(tokens, table):
    tok_flat = tokens.reshape(BFLAT).astype(jnp.int32)
    mesh = plsc.VectorSubcoreMesh(core_axis_name="c", subcore_axis_name="s")
    out = pl.kernel(
        _emb_kernel,
        out_type=jax.ShapeDtypeStruct((BFLAT, EMB), jnp.float32),
        mesh=mesh,
        scratch_types=[
            pltpu.VMEM((PER_W,), jnp.int32),
            pltpu.VMEM((CHUNK, EMB), jnp.float32),
            pltpu.VMEM((CHUNK, EMB), jnp.float32),
            pltpu.SemaphoreType.DMA,
            pltpu.SemaphoreType.DMA,
            pltpu.SemaphoreType.DMA,
            pltpu.SemaphoreType.DMA,
        ],
        compiler_params=pltpu.CompilerParams(use_tc_tiling_on_sc=False),
    )(tok_flat, table)
    return out.reshape(B, L, EMB)
